# baseline (speedup 1.0000x reference)
import functools

import jax
import jax.numpy as jnp
from jax import lax
from jax.experimental import pallas as pl
from jax.experimental.pallas import tpu as pltpu
from jax.experimental.pallas import tpu_sc as plsc

EMB = 64
IDX_CHUNK = 128


def _make_gather(B, n_users, n_movies):
    info = plsc.get_sparse_core_info()
    NC, NS = info.num_cores, info.num_subcores
    NW = NC * NS
    b_per_w = B // NW
    n_chunks = b_per_w // IDX_CHUNK
    mesh = plsc.VectorSubcoreMesh(core_axis_name="c", subcore_axis_name="s")

    @functools.partial(
        pl.kernel,
        mesh=mesh,
        out_type=[
            jax.ShapeDtypeStruct((B, EMB), jnp.float32),
            jax.ShapeDtypeStruct((B, EMB), jnp.float32),
        ],
        scratch_types=[
            pltpu.VMEM((b_per_w,), jnp.int32),
            pltpu.VMEM((b_per_w,), jnp.int32),
            pltpu.VMEM((b_per_w, EMB), jnp.float32),
            pltpu.VMEM((b_per_w, EMB), jnp.float32),
            pltpu.SemaphoreType.DMA,
        ],
        compiler_params=pltpu.CompilerParams(use_tc_tiling_on_sc=False),
    )
    def gather_k(users_h, movies_h, ut_h, mt_h, ue_out, me_out,
                 uidx_v, midx_v, urows_v, mrows_v, sem):
        wid = lax.axis_index("s") * NC + lax.axis_index("c")
        base = wid * b_per_w
        pltpu.sync_copy(users_h.at[pl.ds(base, b_per_w)], uidx_v)
        pltpu.sync_copy(movies_h.at[pl.ds(base, b_per_w)], midx_v)
        copies = []
        for c in range(n_chunks):
            sl = pl.ds(c * IDX_CHUNK, IDX_CHUNK)
            copies.append(pltpu.async_copy(
                ut_h.at[uidx_v.at[sl]], urows_v.at[sl], sem))
            copies.append(pltpu.async_copy(
                mt_h.at[midx_v.at[sl]], mrows_v.at[sl], sem))
        for cp in copies:
            cp.wait()
        pltpu.sync_copy(urows_v, ue_out.at[pl.ds(base, b_per_w)])
        pltpu.sync_copy(mrows_v, me_out.at[pl.ds(base, b_per_w)])

    return gather_k


def _mlp_body(ue_ref, me_ref, w1u_ref, w1m_ref, b1_ref, w2_ref, b2_ref,
              w3_ref, b3_ref, out_ref):
    h = jnp.dot(ue_ref[...], w1u_ref[...], preferred_element_type=jnp.float32)
    h = h + jnp.dot(me_ref[...], w1m_ref[...], preferred_element_type=jnp.float32)
    h = jnp.maximum(h + b1_ref[...], 0.0)
    h = jnp.dot(h, w2_ref[...], preferred_element_type=jnp.float32)
    h = jnp.maximum(h + b2_ref[...], 0.0)
    o = jnp.sum(h * w3_ref[...], axis=1) + b3_ref[0, 0]
    out_ref[...] = o[None, :]


def _mlp(ue, me, w1u, w1m, b1, w2, b2, w3row, b3, blk=2048, interpret=False):
    B = ue.shape[0]
    grid = (B // blk,)
    const = lambda i: (0, 0)
    return pl.pallas_call(
        _mlp_body,
        grid=grid,
        in_specs=[
            pl.BlockSpec((blk, EMB), lambda i: (i, 0)),
            pl.BlockSpec((blk, EMB), lambda i: (i, 0)),
            pl.BlockSpec((EMB, 128), const),
            pl.BlockSpec((EMB, 128), const),
            pl.BlockSpec((1, 128), const),
            pl.BlockSpec((128, 64), const),
            pl.BlockSpec((1, 64), const),
            pl.BlockSpec((1, 64), const),
            pl.BlockSpec((1, 1), const),
        ],
        out_specs=pl.BlockSpec((1, blk), lambda i: (0, i)),
        out_shape=jax.ShapeDtypeStruct((1, B), jnp.float32),
        interpret=interpret,
    )(ue, me, w1u, w1m, b1, w2, b2, w3row, b3)


def kernel(users, movies, user_table, movie_table, W1, b1, W2, b2, W3, b3):
    B = users.shape[0]
    gather_k = _make_gather(B, user_table.shape[0], movie_table.shape[0])
    ue, me = gather_k(users.astype(jnp.int32), movies.astype(jnp.int32),
                      user_table, movie_table)
    out = _mlp(ue, me, W1[:EMB], W1[EMB:], b1.reshape(1, -1), W2,
               b2.reshape(1, -1), W3.reshape(1, -1), b3.reshape(1, 1))
    return out.reshape(B)


# speedup vs baseline: 1.0049x; 1.0049x over previous
import functools

import jax
import jax.numpy as jnp
from jax import lax
from jax.experimental import pallas as pl
from jax.experimental.pallas import tpu as pltpu
from jax.experimental.pallas import tpu_sc as plsc

EMB = 64
WIDE = 2 * EMB
IDX_CHUNK = 128


def _make_gather(B):
    info = plsc.get_sparse_core_info()
    NC, NS = info.num_cores, info.num_subcores
    NW = NC * NS
    b_per_w = B // NW
    n_chunks = b_per_w // IDX_CHUNK
    mesh = plsc.VectorSubcoreMesh(core_axis_name="c", subcore_axis_name="s")

    @functools.partial(
        pl.kernel,
        mesh=mesh,
        out_type=[
            jax.ShapeDtypeStruct((B, WIDE), jnp.float32),
            jax.ShapeDtypeStruct((B, WIDE), jnp.float32),
        ],
        scratch_types=[
            pltpu.VMEM((b_per_w,), jnp.int32),
            pltpu.VMEM((b_per_w,), jnp.int32),
            pltpu.VMEM((2, IDX_CHUNK, WIDE), jnp.float32),
            pltpu.VMEM((2, IDX_CHUNK, WIDE), jnp.float32),
            pltpu.SemaphoreType.DMA,
            pltpu.SemaphoreType.DMA,
            pltpu.SemaphoreType.DMA,
            pltpu.SemaphoreType.DMA,
        ],
    )
    def gather_k(uidx_h, midx_h, ut_h, mt_h, uw_out, mw_out,
                 uidx_v, midx_v, ubuf, mbuf, su0, su1, sm0, sm1):
        wid = lax.axis_index("s") * NC + lax.axis_index("c")
        base = wid * b_per_w
        pltpu.sync_copy(uidx_h.at[pl.ds(base, b_per_w)], uidx_v)
        pltpu.sync_copy(midx_h.at[pl.ds(base, b_per_w)], midx_v)
        sems_u = (su0, su1)
        sems_m = (sm0, sm1)

        def start(c):
            slot = c % 2
            sl = pl.ds(c * IDX_CHUNK, IDX_CHUNK)
            cu = pltpu.async_copy(ut_h.at[uidx_v.at[sl]], ubuf.at[slot],
                                  sems_u[slot])
            cm = pltpu.async_copy(mt_h.at[midx_v.at[sl]], mbuf.at[slot],
                                  sems_m[slot])
            return cu, cm

        pend = start(0)
        for c in range(n_chunks):
            slot = c % 2
            cu, cm = pend
            if c + 1 < n_chunks:
                pend = start(c + 1)
            osl = pl.ds(base + c * IDX_CHUNK, IDX_CHUNK)
            cu.wait()
            pltpu.sync_copy(ubuf.at[slot], uw_out.at[osl])
            cm.wait()
            pltpu.sync_copy(mbuf.at[slot], mw_out.at[osl])

    return gather_k


def _mlp_body(uw_ref, mw_ref, pu_ref, pm_ref, w1u_ref, w1m_ref, b1_ref,
              w2_ref, b2_ref, w3_ref, b3_ref, out_ref):
    uw = uw_ref[...]
    mw = mw_ref[...]
    ue = uw[:, :EMB] + pu_ref[...] * (uw[:, EMB:] - uw[:, :EMB])
    me = mw[:, :EMB] + pm_ref[...] * (mw[:, EMB:] - mw[:, :EMB])
    h = jnp.dot(ue, w1u_ref[...], preferred_element_type=jnp.float32)
    h = h + jnp.dot(me, w1m_ref[...], preferred_element_type=jnp.float32)
    h = jnp.maximum(h + b1_ref[...], 0.0)
    h = jnp.dot(h, w2_ref[...], preferred_element_type=jnp.float32)
    h = jnp.maximum(h + b2_ref[...], 0.0)
    o = jnp.sum(h * w3_ref[...], axis=1) + b3_ref[0, 0]
    out_ref[...] = o[None, :]


def _mlp(uw, mw, pu, pm, w1u, w1m, b1, w2, b2, w3row, b3, blk=2048):
    B = uw.shape[0]
    grid = (B // blk,)
    const = lambda i: (0, 0)
    return pl.pallas_call(
        _mlp_body,
        grid=grid,
        in_specs=[
            pl.BlockSpec((blk, WIDE), lambda i: (i, 0)),
            pl.BlockSpec((blk, WIDE), lambda i: (i, 0)),
            pl.BlockSpec((blk, 1), lambda i: (i, 0)),
            pl.BlockSpec((blk, 1), lambda i: (i, 0)),
            pl.BlockSpec((EMB, 128), const),
            pl.BlockSpec((EMB, 128), const),
            pl.BlockSpec((1, 128), const),
            pl.BlockSpec((128, 64), const),
            pl.BlockSpec((1, 64), const),
            pl.BlockSpec((1, 64), const),
            pl.BlockSpec((1, 1), const),
        ],
        out_specs=pl.BlockSpec((1, blk), lambda i: (0, i)),
        out_shape=jax.ShapeDtypeStruct((1, B), jnp.float32),
    )(uw, mw, pu, pm, w1u, w1m, b1, w2, b2, w3row, b3)


def kernel(users, movies, user_table, movie_table, W1, b1, W2, b2, W3, b3):
    B = users.shape[0]
    u32 = users.astype(jnp.int32)
    m32 = movies.astype(jnp.int32)
    ut2 = user_table.reshape(-1, WIDE)
    mt2 = movie_table.reshape(-1, WIDE)
    gather_k = _make_gather(B)
    uw, mw = gather_k(u32 // 2, m32 // 2, ut2, mt2)
    pu = (u32 & 1).astype(jnp.float32).reshape(B, 1)
    pm = (m32 & 1).astype(jnp.float32).reshape(B, 1)
    out = _mlp(uw, mw, pu, pm, W1[:EMB], W1[EMB:], b1.reshape(1, -1), W2,
               b2.reshape(1, -1), W3.reshape(1, -1), b3.reshape(1, 1))
    return out.reshape(B)


# speedup vs baseline: 1.1656x; 1.1599x over previous
import functools

import jax
import jax.numpy as jnp
from jax import lax
from jax.experimental import pallas as pl
from jax.experimental.pallas import tpu as pltpu
from jax.experimental.pallas import tpu_sc as plsc

EMB = 64
WIDE = 2 * EMB
IDX_CHUNK = 128


PBLK = 1024


def _pair_body(a_ref, b_ref, out_ref):
    eye = jnp.eye(EMB, dtype=jnp.float32)
    dn = (((0,), (0,)), ((), ()))
    at = lax.dot_general(a_ref[...], eye, dn,
                         preferred_element_type=jnp.float32)
    bt = lax.dot_general(b_ref[...], eye, dn,
                         preferred_element_type=jnp.float32)
    out_ref[...] = jnp.concatenate([at, bt], axis=1)


def _pair_table(tt):
    n = tt.shape[1]
    nb = (n + 2 * PBLK - 1) // (2 * PBLK)
    last = (n - 1) // PBLK
    return pl.pallas_call(
        _pair_body,
        grid=(nb,),
        in_specs=[
            pl.BlockSpec((EMB, PBLK),
                         lambda i: (0, jnp.minimum(2 * i, last))),
            pl.BlockSpec((EMB, PBLK),
                         lambda i: (0, jnp.minimum(2 * i + 1, last))),
        ],
        out_specs=pl.BlockSpec((PBLK, WIDE), lambda i: (i, 0)),
        out_shape=jax.ShapeDtypeStruct((nb * PBLK, WIDE), jnp.float32),
        compiler_params=pltpu.CompilerParams(
            fuse_transposed_lhs_in_matmul=True),
    )(tt, tt)


def _make_gather(B):
    info = plsc.get_sparse_core_info()
    NC, NS = info.num_cores, info.num_subcores
    NW = NC * NS
    b_per_w = B // NW
    n_chunks = b_per_w // IDX_CHUNK
    mesh = plsc.VectorSubcoreMesh(core_axis_name="c", subcore_axis_name="s")

    @functools.partial(
        pl.kernel,
        mesh=mesh,
        out_type=[
            jax.ShapeDtypeStruct((B, WIDE), jnp.float32),
            jax.ShapeDtypeStruct((B, WIDE), jnp.float32),
        ],
        scratch_types=[
            pltpu.VMEM((b_per_w,), jnp.int32),
            pltpu.VMEM((b_per_w,), jnp.int32),
            pltpu.VMEM((2, IDX_CHUNK, WIDE), jnp.float32),
            pltpu.VMEM((2, IDX_CHUNK, WIDE), jnp.float32),
            pltpu.SemaphoreType.DMA,
            pltpu.SemaphoreType.DMA,
            pltpu.SemaphoreType.DMA,
            pltpu.SemaphoreType.DMA,
        ],
    )
    def gather_k(uidx_h, midx_h, ut_h, mt_h, uw_out, mw_out,
                 uidx_v, midx_v, ubuf, mbuf, su0, su1, sm0, sm1):
        wid = lax.axis_index("s") * NC + lax.axis_index("c")
        base = wid * b_per_w
        pltpu.sync_copy(uidx_h.at[pl.ds(base, b_per_w)], uidx_v)
        pltpu.sync_copy(midx_h.at[pl.ds(base, b_per_w)], midx_v)
        sems_u = (su0, su1)
        sems_m = (sm0, sm1)

        def start(c):
            slot = c % 2
            sl = pl.ds(c * IDX_CHUNK, IDX_CHUNK)
            cu = pltpu.async_copy(ut_h.at[uidx_v.at[sl]], ubuf.at[slot],
                                  sems_u[slot])
            cm = pltpu.async_copy(mt_h.at[midx_v.at[sl]], mbuf.at[slot],
                                  sems_m[slot])
            return cu, cm

        pend = start(0)
        for c in range(n_chunks):
            slot = c % 2
            cu, cm = pend
            if c + 1 < n_chunks:
                pend = start(c + 1)
            osl = pl.ds(base + c * IDX_CHUNK, IDX_CHUNK)
            cu.wait()
            pltpu.sync_copy(ubuf.at[slot], uw_out.at[osl])
            cm.wait()
            pltpu.sync_copy(mbuf.at[slot], mw_out.at[osl])

    return gather_k


def _mlp_body(uw_ref, mw_ref, pu_ref, pm_ref, w1u_ref, w1m_ref, b1_ref,
              w2_ref, b2_ref, w3_ref, b3_ref, out_ref):
    uw = uw_ref[...]
    mw = mw_ref[...]
    ue = uw[:, :EMB] + pu_ref[...] * (uw[:, EMB:] - uw[:, :EMB])
    me = mw[:, :EMB] + pm_ref[...] * (mw[:, EMB:] - mw[:, :EMB])
    h = jnp.dot(ue, w1u_ref[...], preferred_element_type=jnp.float32)
    h = h + jnp.dot(me, w1m_ref[...], preferred_element_type=jnp.float32)
    h = jnp.maximum(h + b1_ref[...], 0.0)
    h = jnp.dot(h, w2_ref[...], preferred_element_type=jnp.float32)
    h = jnp.maximum(h + b2_ref[...], 0.0)
    o = jnp.sum(h * w3_ref[...], axis=1) + b3_ref[0, 0]
    out_ref[...] = o[None, :]


def _mlp(uw, mw, pu, pm, w1u, w1m, b1, w2, b2, w3row, b3, blk=2048):
    B = uw.shape[0]
    grid = (B // blk,)
    const = lambda i: (0, 0)
    return pl.pallas_call(
        _mlp_body,
        grid=grid,
        in_specs=[
            pl.BlockSpec((blk, WIDE), lambda i: (i, 0)),
            pl.BlockSpec((blk, WIDE), lambda i: (i, 0)),
            pl.BlockSpec((blk, 1), lambda i: (i, 0)),
            pl.BlockSpec((blk, 1), lambda i: (i, 0)),
            pl.BlockSpec((EMB, 128), const),
            pl.BlockSpec((EMB, 128), const),
            pl.BlockSpec((1, 128), const),
            pl.BlockSpec((128, 64), const),
            pl.BlockSpec((1, 64), const),
            pl.BlockSpec((1, 64), const),
            pl.BlockSpec((1, 1), const),
        ],
        out_specs=pl.BlockSpec((1, blk), lambda i: (0, i)),
        out_shape=jax.ShapeDtypeStruct((1, B), jnp.float32),
    )(uw, mw, pu, pm, w1u, w1m, b1, w2, b2, w3row, b3)


def kernel(users, movies, user_table, movie_table, W1, b1, W2, b2, W3, b3):
    B = users.shape[0]
    nu = user_table.shape[0]
    nm = movie_table.shape[0]
    u32 = users.astype(jnp.int32)
    m32 = movies.astype(jnp.int32)
    ut2 = _pair_table(user_table.T)
    mt2 = jnp.concatenate(
        [movie_table[: nm // 2], movie_table[nm // 2 :]], axis=1)
    gather_k = _make_gather(B)
    urow = (u32 // (2 * PBLK)) * PBLK + (u32 & (PBLK - 1))
    uw, mw = gather_k(urow, m32 % (nm // 2), ut2, mt2)
    pu = ((u32 & (2 * PBLK - 1)) >= PBLK).astype(jnp.float32).reshape(B, 1)
    pm = (m32 >= nm // 2).astype(jnp.float32).reshape(B, 1)
    out = _mlp(uw, mw, pu, pm, W1[:EMB], W1[EMB:], b1.reshape(1, -1), W2,
               b2.reshape(1, -1), W3.reshape(1, -1), b3.reshape(1, 1))
    return out.reshape(B)
